# Initial kernel scaffold; baseline (speedup 1.0000x reference)
import jax
import jax.numpy as jnp
from jax.experimental import pallas as pl


def kernel(gh_label, gah_label, p_gh, p_gah, mask):
    raise NotImplementedError("write your pallas kernel here")


# speedup vs baseline: 79.2859x; 79.2859x over previous
import jax
import jax.numpy as jnp
from jax.experimental import pallas as pl
from jax.experimental.pallas import tpu as pltpu

_B = 8
_N = 384 * 384
_SUB = 1152
_LANE = 128
_ROUNDS = 31


def _loss_body(gh_ref, gah_ref, pgh_ref, pgah_ref, m_ref, out_ref, u_ref):
    m = m_ref[...]

    n_pos_l, pos_sum_l, neg_sum_l = [], [], []
    for half, (lab_ref, p_ref) in enumerate(((gh_ref, pgh_ref), (gah_ref, pgah_ref))):
        lab = lab_ref[...]
        d = p_ref[...] - lab
        v = d * d * m
        pm = lab >= 0.1
        pmf = pm.astype(jnp.float32)
        n_pos_l.append(jnp.sum(pmf, axis=(1, 2), keepdims=True))
        pos_sum_l.append(jnp.sum(jnp.where(pm, v, 0.0), axis=(1, 2), keepdims=True))
        neg_sum_l.append(jnp.sum(jnp.where(pm, 0.0, v), axis=(1, 2), keepdims=True))
        u = jnp.where(pm, jnp.int32(-1), jax.lax.bitcast_convert_type(v, jnp.int32))
        u_ref[half * _B:(half + 1) * _B] = u

    n_pos = jnp.concatenate(n_pos_l, axis=0)
    pos_sum = jnp.concatenate(pos_sum_l, axis=0)
    neg_sum = jnp.concatenate(neg_sum_l, axis=0)
    n_neg = jnp.float32(_N) - n_pos
    k = jnp.where(n_pos > 0.0, 3.0 * n_pos, 500.0)

    uu = u_ref[...]
    hi0 = jnp.maximum(jnp.max(uu, axis=(1, 2), keepdims=True), 0)
    lo0 = jnp.zeros_like(hi0)

    def round_fn(_, carry):
        lo, hi = carry
        mid = lo + (hi - lo + 1) // 2
        c = jnp.sum((u_ref[...] >= mid).astype(jnp.float32),
                    axis=(1, 2), keepdims=True)
        ge = c >= k
        return jnp.where(ge, mid, lo), jnp.where(ge, hi, mid - 1)

    lo, _ = jax.lax.fori_loop(0, _ROUNDS, round_fn, (lo0, hi0))
    t = lo

    uu = u_ref[...]
    gt = uu > t
    cnt_gt = jnp.sum(gt.astype(jnp.float32), axis=(1, 2), keepdims=True)
    sum_gt = jnp.sum(
        jnp.where(gt, jax.lax.bitcast_convert_type(uu, jnp.float32), 0.0),
        axis=(1, 2), keepdims=True)
    tval = jax.lax.bitcast_convert_type(t, jnp.float32)
    topk_sum = sum_gt + (k - cnt_gt) * tval

    posi = pos_sum / n_pos
    nega = jnp.where(n_neg < k, neg_sum / n_neg, topk_sum / k)
    row = jnp.where(n_pos > 0.0, posi + nega, topk_sum / 500.0)
    out_ref[...] = jnp.sum(row, axis=0) / jnp.float32(_B)


def _run(gh, gah, pgh, pgah, m, interpret=False):
    out = pl.pallas_call(
        _loss_body,
        out_shape=jax.ShapeDtypeStruct((1, 1), jnp.float32),
        scratch_shapes=[pltpu.VMEM((2 * _B, _SUB, _LANE), jnp.int32)],
        interpret=interpret,
    )(gh, gah, pgh, pgah, m)
    return out[0, 0]


def kernel(gh_label, gah_label, p_gh, p_gah, mask):
    shp = (_B, _SUB, _LANE)
    return _run(gh_label.reshape(shp), gah_label.reshape(shp),
                p_gh.reshape(shp), p_gah.reshape(shp), mask.reshape(shp))
